# Initial kernel scaffold; baseline (speedup 1.0000x reference)
import functools

import jax
import jax.numpy as jnp
from jax import lax
from jax.experimental import pallas as pl
from jax.experimental.pallas import tpu as pltpu
from jax.experimental.pallas import tpu_sc as plsc

_NC = 2
_NS = 16
_NW = _NC * _NS
_CHUNK = 128


def _sc_mesh():
    return plsc.VectorSubcoreMesh(core_axis_name="c", subcore_axis_name="s")


def _make_deg_kernel(n_acc, cpt):
    rpt = n_acc // _NS

    @functools.partial(
        pl.kernel,
        out_type=jax.ShapeDtypeStruct((_NC, n_acc, 16), jnp.float32),
        mesh=_sc_mesh(),
        scratch_types=[
            pltpu.VMEM((cpt, _CHUNK), jnp.int32),
            pltpu.VMEM((_CHUNK, 16), jnp.float32),
            pltpu.VMEM_SHARED((n_acc, 16), jnp.float32),
        ],
    )
    def deg_kernel(dst_hbm, ones_hbm, zeros_hbm, out_hbm, dst_v, ones_v, acc_sh):
        c = lax.axis_index("c")
        s = lax.axis_index("s")
        w = s * _NC + c
        pltpu.sync_copy(zeros_hbm, acc_sh.at[pl.ds(s * rpt, rpt)])
        pltpu.sync_copy(ones_hbm, ones_v)
        pltpu.sync_copy(dst_hbm.at[w], dst_v)
        plsc.subcore_barrier()

        def body(j, carry):
            pltpu.sync_copy(ones_v, acc_sh.at[dst_v.at[j]], add=True)
            return carry

        lax.fori_loop(0, cpt, body, 0)
        plsc.subcore_barrier()
        pltpu.sync_copy(acc_sh.at[pl.ds(s * rpt, rpt)],
                        out_hbm.at[c, pl.ds(s * rpt, rpt)])

    return deg_kernel


def _make_agg_kernel(n_acc, cpt, d):
    rpt = n_acc // _NS

    @functools.partial(
        pl.kernel,
        out_type=jax.ShapeDtypeStruct((_NC, n_acc, d), jnp.float32),
        mesh=_sc_mesh(),
        scratch_types=[
            pltpu.VMEM((cpt, _CHUNK), jnp.int32),
            pltpu.VMEM((cpt, _CHUNK), jnp.int32),
            pltpu.VMEM((2, _CHUNK, d), jnp.float32),
            pltpu.VMEM_SHARED((n_acc, d), jnp.float32),
            pltpu.SemaphoreType.DMA,
        ],
    )
    def agg_kernel(src_hbm, dst_hbm, tbl_hbm, zeros_hbm, out_hbm,
                   src_v, dst_v, rows_v, acc_sh, gsem):
        c = lax.axis_index("c")
        s = lax.axis_index("s")
        w = s * _NC + c
        pltpu.sync_copy(zeros_hbm, acc_sh.at[pl.ds(s * rpt, rpt)])
        pltpu.sync_copy(src_hbm.at[w], src_v)
        pltpu.sync_copy(dst_hbm.at[w], dst_v)
        plsc.subcore_barrier()

        pltpu.async_copy(tbl_hbm.at[src_v.at[0]], rows_v.at[0], gsem)

        def body(j, carry):
            b = lax.rem(j, 2)
            pltpu.make_async_copy(tbl_hbm.at[src_v.at[j]], rows_v.at[b],
                                  gsem).wait()

            @pl.when(j + 1 < cpt)
            def _():
                pltpu.async_copy(tbl_hbm.at[src_v.at[j + 1]],
                                 rows_v.at[1 - b], gsem)

            pltpu.sync_copy(rows_v.at[b], acc_sh.at[dst_v.at[j]], add=True)
            return carry

        lax.fori_loop(0, cpt, body, 0)
        plsc.subcore_barrier()
        pltpu.sync_copy(acc_sh.at[pl.ds(s * rpt, rpt)],
                        out_hbm.at[c, pl.ds(s * rpt, rpt)])

    return agg_kernel


def _tc_layer1(x_pad, w1, degp):
    np_, f_in = x_pad.shape
    f_hid = w1.shape[1]

    def body(x_ref, w_ref, degp_ref, h_ref, dinv_ref):
        degp = degp_ref[...]
        deg = degp[0, :, 0] + degp[1, :, 0]
        dinv = jnp.where(deg > 0.0, lax.rsqrt(jnp.maximum(deg, 1e-12)), 0.0)
        h = jnp.dot(x_ref[...], w_ref[...], preferred_element_type=jnp.float32)
        h_ref[...] = h * dinv[:, None]
        dinv_ref[...] = jnp.broadcast_to(dinv[:, None], dinv_ref.shape)

    return pl.pallas_call(
        body,
        out_shape=[jax.ShapeDtypeStruct((np_, f_hid), jnp.float32),
                   jax.ShapeDtypeStruct((np_, 16), jnp.float32)],
    )(x_pad, w1, degp)


def _tc_mid(aggp, dinvb, b1, w2):
    np_ = aggp.shape[1]
    f_out = w2.shape[1]

    def body(aggp_ref, dinv_ref, b1_ref, w2_ref, out_ref):
        agg = aggp_ref[0] + aggp_ref[1]
        dinv = dinv_ref[...][:, :1]
        hmid = jnp.maximum(agg * dinv + b1_ref[...], 0.0)
        h2 = jnp.dot(hmid, w2_ref[...], preferred_element_type=jnp.float32)
        out_ref[...] = h2 * dinv

    return pl.pallas_call(
        body,
        out_shape=jax.ShapeDtypeStruct((np_, f_out), jnp.float32),
    )(aggp, dinvb, b1, w2)


def _tc_final(aggp, dinvb, b2):
    np_, f_out = aggp.shape[1], aggp.shape[2]

    def body(aggp_ref, dinv_ref, b2_ref, out_ref):
        agg = aggp_ref[0] + aggp_ref[1]
        dinv = dinv_ref[...][:, :1]
        o = agg * dinv + b2_ref[...]
        m = jnp.max(o, axis=1, keepdims=True)
        ex = jnp.exp(o - m)
        lse = jnp.log(jnp.sum(ex, axis=1, keepdims=True)) + m
        out_ref[...] = o - lse

    return pl.pallas_call(
        body,
        out_shape=jax.ShapeDtypeStruct((np_, f_out), jnp.float32),
    )(aggp, dinvb, b2)


def kernel(x, edge_index, W1, b1, W2, b2):
    n, f_in = x.shape
    e = edge_index.shape[1]
    f_hid = W1.shape[1]
    f_out = W2.shape[1]

    cpt = -(-e // (_NW * _CHUNK))
    e_pad = _NW * cpt * _CHUNK
    n_acc = -(-(n + 1) // _NS) * _NS
    rpt = n_acc // _NS

    src = edge_index[0]
    dst = edge_index[1]
    pad = e_pad - e
    src_p = jnp.concatenate(
        [src, jnp.zeros((pad,), jnp.int32)]).reshape(_NW, cpt, _CHUNK)
    dst_p = jnp.concatenate(
        [dst, jnp.full((pad,), n, jnp.int32)]).reshape(_NW, cpt, _CHUNK)
    x_pad = jnp.pad(x, ((0, n_acc - n), (0, 0)))

    ones16 = jnp.ones((_CHUNK, 16), jnp.float32)
    zeros16 = jnp.zeros((rpt, 16), jnp.float32)
    zeros_hid = jnp.zeros((rpt, f_hid), jnp.float32)
    zeros_out = jnp.zeros((rpt, f_out), jnp.float32)

    degp = _make_deg_kernel(n_acc, cpt)(dst_p, ones16, zeros16)
    h1, dinvb = _tc_layer1(x_pad, W1, degp)
    agg1p = _make_agg_kernel(n_acc, cpt, f_hid)(src_p, dst_p, h1, zeros_hid)
    h2 = _tc_mid(agg1p, dinvb, b1.reshape(1, f_hid), W2)
    agg2p = _make_agg_kernel(n_acc, cpt, f_out)(src_p, dst_p, h2, zeros_out)
    out = _tc_final(agg2p, dinvb, b2.reshape(1, f_out))
    return out[:n]


# speedup vs baseline: 20.4495x; 20.4495x over previous
import functools

import jax
import jax.numpy as jnp
from jax import lax
from jax.experimental import pallas as pl
from jax.experimental.pallas import tpu as pltpu
from jax.experimental.pallas import tpu_sc as plsc

_NC = 2
_NS = 16
_NW = _NC * _NS
_CHUNK = 128


def _sc_mesh():
    return plsc.VectorSubcoreMesh(core_axis_name="c", subcore_axis_name="s")


def _make_deg_kernel(n_acc, cpt):
    rpt = n_acc // _NS

    @functools.partial(
        pl.kernel,
        out_type=jax.ShapeDtypeStruct((_NC, n_acc, 16), jnp.float32),
        mesh=_sc_mesh(),
        compiler_params=pltpu.CompilerParams(use_tc_tiling_on_sc=False),
        scratch_types=[
            pltpu.VMEM((cpt, _CHUNK), jnp.int32),
            pltpu.VMEM((_CHUNK, 16), jnp.float32),
            pltpu.VMEM_SHARED((n_acc, 16), jnp.float32),
        ],
    )
    def deg_kernel(dst_hbm, ones_hbm, zeros_hbm, out_hbm, dst_v, ones_v, acc_sh):
        c = lax.axis_index("c")
        s = lax.axis_index("s")
        w = s * _NC + c
        pltpu.sync_copy(zeros_hbm, acc_sh.at[pl.ds(s * rpt, rpt)])
        pltpu.sync_copy(ones_hbm, ones_v)
        pltpu.sync_copy(dst_hbm.at[w], dst_v)
        plsc.subcore_barrier()

        def body(j, carry):
            pltpu.sync_copy(ones_v, acc_sh.at[dst_v.at[j]], add=True)
            return carry

        lax.fori_loop(0, cpt, body, 0)
        plsc.subcore_barrier()
        pltpu.sync_copy(acc_sh.at[pl.ds(s * rpt, rpt)],
                        out_hbm.at[c, pl.ds(s * rpt, rpt)])

    return deg_kernel


def _make_agg_split_kernel(n_acc, cpt, dh):
    rpt = n_acc // _NS

    @functools.partial(
        pl.kernel,
        out_type=jax.ShapeDtypeStruct((_NC, n_acc, dh), jnp.float32),
        mesh=_sc_mesh(),
        compiler_params=pltpu.CompilerParams(use_tc_tiling_on_sc=False),
        scratch_types=[
            pltpu.VMEM((cpt, _CHUNK), jnp.int32),
            pltpu.VMEM((cpt, _CHUNK), jnp.int32),
            pltpu.VMEM((2, _CHUNK, dh), jnp.float32),
            pltpu.VMEM_SHARED((n_acc, dh), jnp.float32),
            pltpu.SemaphoreType.DMA,
        ],
    )
    def agg_kernel(src_hbm, dst_hbm, tbl_hbm, zeros_hbm, out_hbm,
                   src_v, dst_v, rows_v, acc_sh, gsem):
        c = lax.axis_index("c")
        s = lax.axis_index("s")
        tbl = tbl_hbm.at[c]
        pltpu.sync_copy(zeros_hbm, acc_sh.at[pl.ds(s * rpt, rpt)])
        pltpu.sync_copy(src_hbm.at[s], src_v)
        pltpu.sync_copy(dst_hbm.at[s], dst_v)
        plsc.subcore_barrier()

        pltpu.async_copy(tbl.at[src_v.at[0]], rows_v.at[0], gsem)

        def body(j, carry):
            b = lax.rem(j, 2)
            pltpu.make_async_copy(tbl.at[src_v.at[j]], rows_v.at[b],
                                  gsem).wait()

            @pl.when(j + 1 < cpt)
            def _():
                pltpu.async_copy(tbl.at[src_v.at[j + 1]],
                                 rows_v.at[1 - b], gsem)

            pltpu.sync_copy(rows_v.at[b], acc_sh.at[dst_v.at[j]], add=True)
            return carry

        lax.fori_loop(0, cpt, body, 0)
        plsc.subcore_barrier()
        pltpu.sync_copy(acc_sh.at[pl.ds(s * rpt, rpt)],
                        out_hbm.at[c, pl.ds(s * rpt, rpt)])

    return agg_kernel


def _make_agg_kernel(n_acc, cpt, d):
    rpt = n_acc // _NS

    @functools.partial(
        pl.kernel,
        out_type=jax.ShapeDtypeStruct((_NC, n_acc, d), jnp.float32),
        mesh=_sc_mesh(),
        compiler_params=pltpu.CompilerParams(use_tc_tiling_on_sc=False),
        scratch_types=[
            pltpu.VMEM((cpt, _CHUNK), jnp.int32),
            pltpu.VMEM((cpt, _CHUNK), jnp.int32),
            pltpu.VMEM((2, _CHUNK, d), jnp.float32),
            pltpu.VMEM_SHARED((n_acc, d), jnp.float32),
            pltpu.SemaphoreType.DMA,
        ],
    )
    def agg_kernel(src_hbm, dst_hbm, tbl_hbm, zeros_hbm, out_hbm,
                   src_v, dst_v, rows_v, acc_sh, gsem):
        c = lax.axis_index("c")
        s = lax.axis_index("s")
        w = s * _NC + c
        pltpu.sync_copy(zeros_hbm, acc_sh.at[pl.ds(s * rpt, rpt)])
        pltpu.sync_copy(src_hbm.at[w], src_v)
        pltpu.sync_copy(dst_hbm.at[w], dst_v)
        plsc.subcore_barrier()

        pltpu.async_copy(tbl_hbm.at[src_v.at[0]], rows_v.at[0], gsem)

        def body(j, carry):
            b = lax.rem(j, 2)
            pltpu.make_async_copy(tbl_hbm.at[src_v.at[j]], rows_v.at[b],
                                  gsem).wait()

            @pl.when(j + 1 < cpt)
            def _():
                pltpu.async_copy(tbl_hbm.at[src_v.at[j + 1]],
                                 rows_v.at[1 - b], gsem)

            pltpu.sync_copy(rows_v.at[b], acc_sh.at[dst_v.at[j]], add=True)
            return carry

        lax.fori_loop(0, cpt, body, 0)
        plsc.subcore_barrier()
        pltpu.sync_copy(acc_sh.at[pl.ds(s * rpt, rpt)],
                        out_hbm.at[c, pl.ds(s * rpt, rpt)])

    return agg_kernel


def _tc_layer1(x_pad, w1, degp):
    np_, f_in = x_pad.shape
    f_hid = w1.shape[1]

    dh = f_hid // 2

    def body(x_ref, w_ref, degp_ref, h_ref, dinv_ref):
        degp = degp_ref[...]
        deg = degp[0, :, 0] + degp[1, :, 0]
        dinv = jnp.where(deg > 0.0, lax.rsqrt(jnp.maximum(deg, 1e-12)), 0.0)
        h = jnp.dot(x_ref[...], w_ref[...], preferred_element_type=jnp.float32)
        h = h * dinv[:, None]
        h_ref[0] = h[:, :dh]
        h_ref[1] = h[:, dh:]
        dinv_ref[...] = jnp.broadcast_to(dinv[:, None], dinv_ref.shape)

    return pl.pallas_call(
        body,
        out_shape=[jax.ShapeDtypeStruct((2, np_, dh), jnp.float32),
                   jax.ShapeDtypeStruct((np_, 16), jnp.float32)],
    )(x_pad, w1, degp)


def _tc_mid(aggp, dinvb, b1, w2):
    np_ = aggp.shape[1]
    f_out = w2.shape[1]

    def body(aggp_ref, dinv_ref, b1_ref, w2_ref, out_ref):
        agg = jnp.concatenate([aggp_ref[0], aggp_ref[1]], axis=1)
        dinv = dinv_ref[...][:, :1]
        hmid = jnp.maximum(agg * dinv + b1_ref[...], 0.0)
        h2 = jnp.dot(hmid, w2_ref[...], preferred_element_type=jnp.float32)
        out_ref[...] = h2 * dinv

    return pl.pallas_call(
        body,
        out_shape=jax.ShapeDtypeStruct((np_, f_out), jnp.float32),
    )(aggp, dinvb, b1, w2)


def _tc_final(aggp, dinvb, b2):
    np_, f_out = aggp.shape[1], aggp.shape[2]

    def body(aggp_ref, dinv_ref, b2_ref, out_ref):
        agg = aggp_ref[0] + aggp_ref[1]
        dinv = dinv_ref[...][:, :1]
        o = agg * dinv + b2_ref[...]
        m = jnp.max(o, axis=1, keepdims=True)
        ex = jnp.exp(o - m)
        lse = jnp.log(jnp.sum(ex, axis=1, keepdims=True)) + m
        out_ref[...] = o - lse

    return pl.pallas_call(
        body,
        out_shape=jax.ShapeDtypeStruct((np_, f_out), jnp.float32),
    )(aggp, dinvb, b2)


def kernel(x, edge_index, W1, b1, W2, b2):
    n, f_in = x.shape
    e = edge_index.shape[1]
    f_hid = W1.shape[1]
    f_out = W2.shape[1]

    cpt = -(-e // (_NW * _CHUNK))
    e_pad = _NW * cpt * _CHUNK
    cpt2 = -(-e // (_NS * _CHUNK))
    e_pad2 = _NS * cpt2 * _CHUNK
    n_acc = -(-(n + 1) // (_NS * 8)) * (_NS * 8)
    rpt = n_acc // _NS
    dh = f_hid // 2

    src = edge_index[0]
    dst = edge_index[1]

    def _lay(v, fill, nw, k):
        pad_amt = nw * k * _CHUNK - e
        return jnp.concatenate(
            [v, jnp.full((pad_amt,), fill, jnp.int32)]).reshape(nw, k, _CHUNK)

    src_p = _lay(src, 0, _NW, cpt)
    dst_p = _lay(dst, n, _NW, cpt)
    src_q = _lay(src, 0, _NS, cpt2)
    dst_q = _lay(dst, n, _NS, cpt2)
    x_pad = jnp.pad(x, ((0, n_acc - n), (0, 0)))

    ones16 = jnp.ones((_CHUNK, 16), jnp.float32)
    zeros16 = jnp.zeros((rpt, 16), jnp.float32)
    zeros_hid = jnp.zeros((rpt, dh), jnp.float32)
    zeros_out = jnp.zeros((rpt, f_out), jnp.float32)

    degp = _make_deg_kernel(n_acc, cpt)(dst_p, ones16, zeros16)
    h1s, dinvb = _tc_layer1(x_pad, W1, degp)
    agg1s = _make_agg_split_kernel(n_acc, cpt2, dh)(src_q, dst_q, h1s,
                                                    zeros_hid)
    h2 = _tc_mid(agg1s, dinvb, b1.reshape(1, f_hid), W2)
    agg2p = _make_agg_kernel(n_acc, cpt, f_out)(src_p, dst_p, h2, zeros_out)
    out = _tc_final(agg2p, dinvb, b2.reshape(1, f_out))
    return out[:n]


# speedup vs baseline: 27.4020x; 1.3400x over previous
import functools

import jax
import jax.numpy as jnp
from jax import lax
from jax.experimental import pallas as pl
from jax.experimental.pallas import tpu as pltpu
from jax.experimental.pallas import tpu_sc as plsc

_NC = 2
_NS = 16
_NW = _NC * _NS
_CHUNK = 128


def _sc_mesh():
    return plsc.VectorSubcoreMesh(core_axis_name="c", subcore_axis_name="s")


def _make_deg_kernel(n_acc, cpt):
    rpt = n_acc // _NS

    @functools.partial(
        pl.kernel,
        out_type=jax.ShapeDtypeStruct((_NC, n_acc, 16), jnp.float32),
        mesh=_sc_mesh(),
        compiler_params=pltpu.CompilerParams(use_tc_tiling_on_sc=False),
        scratch_types=[
            pltpu.VMEM((cpt, _CHUNK), jnp.int32),
            pltpu.VMEM((_CHUNK, 16), jnp.float32),
            pltpu.VMEM_SHARED((n_acc, 16), jnp.float32),
        ],
    )
    def deg_kernel(dst_hbm, ones_hbm, zeros_hbm, out_hbm, dst_v, ones_v, acc_sh):
        c = lax.axis_index("c")
        s = lax.axis_index("s")
        w = s * _NC + c
        pltpu.sync_copy(zeros_hbm, acc_sh.at[pl.ds(s * rpt, rpt)])
        pltpu.sync_copy(ones_hbm, ones_v)
        pltpu.sync_copy(dst_hbm.at[w], dst_v)
        plsc.subcore_barrier()

        def body(j, carry):
            pltpu.sync_copy(ones_v, acc_sh.at[dst_v.at[j]], add=True)
            return carry

        lax.fori_loop(0, cpt, body, 0)
        plsc.subcore_barrier()
        pltpu.sync_copy(acc_sh.at[pl.ds(s * rpt, rpt)],
                        out_hbm.at[c, pl.ds(s * rpt, rpt)])

    return deg_kernel


_NBUF = 4


def _make_agg_kernel(n_acc, cpt, d, split):
    rpt = n_acc // _NS

    @functools.partial(
        pl.kernel,
        out_type=jax.ShapeDtypeStruct((_NC, n_acc, d), jnp.float32),
        mesh=_sc_mesh(),
        compiler_params=pltpu.CompilerParams(use_tc_tiling_on_sc=False),
        scratch_types=[
            pltpu.VMEM((cpt, _CHUNK), jnp.int32),
            pltpu.VMEM((cpt, _CHUNK), jnp.int32),
            pltpu.VMEM((_NBUF, _CHUNK, d), jnp.float32),
            pltpu.VMEM_SHARED((n_acc, d), jnp.float32),
        ] + [pltpu.SemaphoreType.DMA] * _NBUF,
    )
    def agg_kernel(src_hbm, dst_hbm, tbl_hbm, zeros_hbm, out_hbm,
                   src_v, dst_v, rows_v, acc_sh, *gsem):
        c = lax.axis_index("c")
        s = lax.axis_index("s")
        if split:
            w = s
            tbl = tbl_hbm.at[c]
        else:
            w = s * _NC + c
            tbl = tbl_hbm
        pltpu.sync_copy(zeros_hbm, acc_sh.at[pl.ds(s * rpt, rpt)])
        pltpu.sync_copy(src_hbm.at[w], src_v)
        pltpu.sync_copy(dst_hbm.at[w], dst_v)
        plsc.subcore_barrier()

        for b in range(_NBUF):
            pltpu.async_copy(tbl.at[src_v.at[b]], rows_v.at[b], gsem[b])

        ngroups = -(-cpt // _NBUF)

        def group(gi, carry):
            for b in range(_NBUF):
                j = gi * _NBUF + b

                @pl.when(j < cpt)
                def _():
                    pltpu.make_async_copy(tbl.at[src_v.at[j]], rows_v.at[b],
                                          gsem[b]).wait()
                    pltpu.sync_copy(rows_v.at[b], acc_sh.at[dst_v.at[j]],
                                    add=True)

                    @pl.when(j + _NBUF < cpt)
                    def _():
                        pltpu.async_copy(tbl.at[src_v.at[j + _NBUF]],
                                         rows_v.at[b], gsem[b])

            return carry

        lax.fori_loop(0, ngroups, group, 0)
        plsc.subcore_barrier()
        pltpu.sync_copy(acc_sh.at[pl.ds(s * rpt, rpt)],
                        out_hbm.at[c, pl.ds(s * rpt, rpt)])

    return agg_kernel


def _tc_layer1(x_pad, w1, degp):
    np_, f_in = x_pad.shape
    f_hid = w1.shape[1]

    dh = f_hid // 2

    def body(x_ref, w_ref, degp_ref, h_ref, dinv_ref):
        degp = degp_ref[...]
        deg = degp[0, :, 0] + degp[1, :, 0]
        dinv = jnp.where(deg > 0.0, lax.rsqrt(jnp.maximum(deg, 1e-12)), 0.0)
        h = jnp.dot(x_ref[...], w_ref[...], preferred_element_type=jnp.float32)
        h = h * dinv[:, None]
        h_ref[0] = h[:, :dh]
        h_ref[1] = h[:, dh:]
        dinv_ref[...] = jnp.broadcast_to(dinv[:, None], dinv_ref.shape)

    return pl.pallas_call(
        body,
        out_shape=[jax.ShapeDtypeStruct((2, np_, dh), jnp.float32),
                   jax.ShapeDtypeStruct((np_, 16), jnp.float32)],
    )(x_pad, w1, degp)


def _tc_mid(aggp, dinvb, b1, w2):
    np_ = aggp.shape[1]
    f_out = w2.shape[1]

    def body(aggp_ref, dinv_ref, b1_ref, w2_ref, out_ref):
        agg = jnp.concatenate([aggp_ref[0], aggp_ref[1]], axis=1)
        dinv = dinv_ref[...][:, :1]
        hmid = jnp.maximum(agg * dinv + b1_ref[...], 0.0)
        h2 = jnp.dot(hmid, w2_ref[...], preferred_element_type=jnp.float32)
        out_ref[...] = h2 * dinv

    return pl.pallas_call(
        body,
        out_shape=jax.ShapeDtypeStruct((np_, f_out), jnp.float32),
    )(aggp, dinvb, b1, w2)


def _tc_final(aggp, dinvb, b2):
    np_, f_out = aggp.shape[1], aggp.shape[2]

    def body(aggp_ref, dinv_ref, b2_ref, out_ref):
        agg = aggp_ref[0] + aggp_ref[1]
        dinv = dinv_ref[...][:, :1]
        o = agg * dinv + b2_ref[...]
        m = jnp.max(o, axis=1, keepdims=True)
        ex = jnp.exp(o - m)
        lse = jnp.log(jnp.sum(ex, axis=1, keepdims=True)) + m
        out_ref[...] = o - lse

    return pl.pallas_call(
        body,
        out_shape=jax.ShapeDtypeStruct((np_, f_out), jnp.float32),
    )(aggp, dinvb, b2)


def kernel(x, edge_index, W1, b1, W2, b2):
    n, f_in = x.shape
    e = edge_index.shape[1]
    f_hid = W1.shape[1]
    f_out = W2.shape[1]

    cpt = -(-e // (_NW * _CHUNK))
    e_pad = _NW * cpt * _CHUNK
    cpt2 = -(-e // (_NS * _CHUNK))
    e_pad2 = _NS * cpt2 * _CHUNK
    n_acc = -(-(n + 1) // (_NS * 8)) * (_NS * 8)
    rpt = n_acc // _NS
    dh = f_hid // 2

    src = edge_index[0]
    dst = edge_index[1]

    def _lay(v, fill, nw, k):
        pad_amt = nw * k * _CHUNK - e
        return jnp.concatenate(
            [v, jnp.full((pad_amt,), fill, jnp.int32)]).reshape(nw, k, _CHUNK)

    src_p = _lay(src, 0, _NW, cpt)
    dst_p = _lay(dst, n, _NW, cpt)
    src_q = _lay(src, 0, _NS, cpt2)
    dst_q = _lay(dst, n, _NS, cpt2)
    x_pad = jnp.pad(x, ((0, n_acc - n), (0, 0)))

    ones16 = jnp.ones((_CHUNK, 16), jnp.float32)
    zeros16 = jnp.zeros((rpt, 16), jnp.float32)
    zeros_hid = jnp.zeros((rpt, dh), jnp.float32)
    zeros_out = jnp.zeros((rpt, f_out), jnp.float32)

    degp = _make_deg_kernel(n_acc, cpt)(dst_p, ones16, zeros16)
    h1s, dinvb = _tc_layer1(x_pad, W1, degp)
    agg1s = _make_agg_kernel(n_acc, cpt2, dh, split=True)(src_q, dst_q, h1s,
                                                          zeros_hid)
    h2 = _tc_mid(agg1s, dinvb, b1.reshape(1, f_hid), W2)
    agg2p = _make_agg_kernel(n_acc, cpt, f_out, split=False)(src_p, dst_p, h2,
                                                             zeros_out)
    out = _tc_final(agg2p, dinvb, b2.reshape(1, f_out))
    return out[:n]
